# baseline (speedup 1.0000x reference)
import functools

import jax
import jax.numpy as jnp
from jax import lax
from jax.experimental import pallas as pl
from jax.experimental.pallas import tpu as pltpu
from jax.experimental.pallas import tpu_sc as plsc

_NW = 32
_NBUF = 4
_CB = 128
_L = 16


@functools.partial(jax.jit, static_argnums=(2,))
def _embed(x, table2, v):
    b, h = x.shape
    d = 32
    b_per_w = b // _NW
    nch = b_per_w // _CB
    ngrp = (h + _L - 1) // _L
    gstride = ngrp * _L
    gmax = v // 4 - 1
    mesh = plsc.VectorSubcoreMesh(core_axis_name="c", subcore_axis_name="s")

    @functools.partial(
        pl.kernel,
        out_type=jax.ShapeDtypeStruct((b, h, d), jnp.float32),
        mesh=mesh,
        scratch_types=[
            pltpu.VMEM((_CB, h), jnp.int32),
            pltpu.VMEM((_CB * gstride,), jnp.int32),
            pltpu.VMEM((_NBUF, h, 128), jnp.float32),
            pltpu.VMEM((_NBUF, h, d), jnp.float32),
            pltpu.SemaphoreType.DMA((_NBUF,)),
            pltpu.SemaphoreType.DMA((_NBUF,)),
        ],
        compiler_params=pltpu.CompilerParams(
            use_tc_tiling_on_sc=True, needs_layout_passes=False),
    )
    def k(x_hbm, tab_hbm, out_hbm, idx_v, grp_v, stag_v, slab_v, gsem, osem):
        wid = lax.axis_index("s") * 2 + lax.axis_index("c")
        wb0 = wid * b_per_w
        iota = lax.iota(jnp.int32, _L)

        def read_idx(ib, g):
            rows = g * _L + iota
            if (g + 1) * _L > h:
                rows = jnp.minimum(rows, jnp.int32(h - 1))
            return plsc.load_gather(idx_v.at[ib], [rows])

        def fire_gather(ib, s):
            pltpu.async_copy(
                tab_hbm.at[grp_v.at[pl.ds(ib * gstride, h)]],
                stag_v.at[s], gsem.at[s])

        def wait_gather(s):
            pltpu.make_async_copy(
                tab_hbm.at[grp_v.at[pl.ds(0, h)]],
                stag_v.at[s], gsem.at[s]).wait()

        def fire_write(bb, s):
            pltpu.async_copy(slab_v.at[s], out_hbm.at[bb], osem.at[s])

        def wait_write(s):
            pltpu.make_async_copy(
                slab_v.at[s], out_hbm.at[wb0], osem.at[s]).wait()

        for ic in range(nch):
            cb0 = wb0 + ic * _CB
            pltpu.sync_copy(x_hbm.at[pl.ds(cb0, _CB)], idx_v)

            @pl.loop(0, _CB)
            def _(ib):
                for g in range(ngrp):
                    iv = read_idx(ib, g)
                    gv = jnp.minimum(
                        jnp.maximum(iv, 0) >> 2, jnp.int32(gmax))
                    grp_v[pl.ds(ib * gstride + g * _L, _L)] = gv

            for s in range(_NBUF):
                fire_gather(s, s)

            @pl.loop(0, _CB, step=_NBUF)
            def _(j):
                for s in range(_NBUF):
                    ib = j + s
                    wait_gather(s)

                    @pl.when(ib >= _NBUF)
                    def _():
                        wait_write(s)

                    for g in range(ngrp):
                        rows = g * _L + iota
                        if (g + 1) * _L > h:
                            m = rows < h
                            rows = jnp.minimum(rows, jnp.int32(h - 1))
                        else:
                            m = None
                        iv = read_idx(ib, g)
                        qoff = (iv & 3) << 5
                        for kk in range(d):
                            lanes = qoff + kk
                            vals = plsc.load_gather(
                                stag_v.at[s], [rows, lanes], mask=m)
                            plsc.store_scatter(
                                slab_v.at[s],
                                [rows, jnp.full((_L,), kk, jnp.int32)],
                                vals, mask=m)

                    fire_write(cb0 + ib, s)

                    @pl.when(ib + _NBUF < _CB)
                    def _():
                        fire_gather(ib + _NBUF, s)

            for s in range(_NBUF):
                wait_write(s)

    return k(x, table2)


def kernel(x, table):
    v, d = table.shape
    table2 = table.reshape(v // 4, d * 4)
    return _embed(x.astype(jnp.int32), table2, v)


# speedup vs baseline: 4.3269x; 4.3269x over previous
import functools

import jax
import jax.numpy as jnp
from jax import lax
from jax.experimental import pallas as pl
from jax.experimental.pallas import tpu as pltpu
from jax.experimental.pallas import tpu_sc as plsc

_NSC = 2
_NSUB = 16


@jax.jit
def _embed_t(x_t, table_t):
    h, b = x_t.shape
    d, v = table_t.shape
    kpc = d // _NSC
    nj = (h + _NSUB - 1) // _NSUB
    mesh = plsc.VectorSubcoreMesh(core_axis_name="c", subcore_axis_name="s")

    @functools.partial(
        pl.kernel,
        out_type=jax.ShapeDtypeStruct((h, d, b), jnp.float32),
        mesh=mesh,
        scratch_types=(
            [pltpu.VMEM((b,), jnp.int32) for _ in range(2)]
            + [pltpu.VMEM((b,), jnp.float32) for _ in range(2)]
            + [pltpu.VMEM_SHARED((v,), jnp.float32)]
            + [pltpu.SemaphoreType.DMA,
               pltpu.SemaphoreType.DMA((2,)),
               pltpu.SemaphoreType.DMA((2,)),
               pltpu.SemaphoreType.DMA((2,))]
        ),
        compiler_params=pltpu.CompilerParams(use_tc_tiling_on_sc=True),
    )
    def kern(xt_hbm, tab_hbm, out_hbm, *refs):
        idx_bufs = refs[:2]
        gbufs = refs[2:4]
        sp = refs[4]
        stsem, issem, gsem, osem = refs[5:]
        cid = lax.axis_index("c")
        sid = lax.axis_index("s")
        k0 = cid * kpc

        @pl.loop(0, kpc)
        def _(kk):
            k = k0 + kk

            for j in range(2):
                pltpu.async_copy(
                    xt_hbm.at[sid + j * _NSUB], idx_bufs[j], issem.at[j])

            @pl.when(sid == 0)
            def _():
                pltpu.async_copy(tab_hbm.at[k], sp, stsem)
                pltpu.make_async_copy(tab_hbm.at[k], sp, stsem).wait()
            plsc.subcore_barrier()

            for j in range(nj):
                p = j % 2
                hj = sid + j * _NSUB
                have = hj < h if j * _NSUB + _NSUB > h else True
                hj2 = hj + 2 * _NSUB

                @pl.when(have)
                def _():
                    pltpu.make_async_copy(
                        xt_hbm.at[hj], idx_bufs[p], issem.at[p]).wait()

                    @pl.when(jnp.logical_or(kk > 0, j >= 2))
                    def _():
                        pltpu.make_async_copy(
                            gbufs[p], out_hbm.at[hj, k], osem.at[p]).wait()
                    pltpu.async_copy(
                        sp.at[idx_bufs[p]], gbufs[p], gsem.at[p])
                    pltpu.make_async_copy(
                        sp.at[idx_bufs[p]], gbufs[p], gsem.at[p]).wait()
                    pltpu.async_copy(gbufs[p], out_hbm.at[hj, k], osem.at[p])
                    if j + 2 < nj:
                        @pl.when(hj2 < h)
                        def _():
                            pltpu.async_copy(
                                xt_hbm.at[hj2], idx_bufs[p], issem.at[p])

            plsc.subcore_barrier()

        for p in (0, 1):
            pltpu.make_async_copy(
                gbufs[p], out_hbm.at[0, k0], osem.at[p]).wait()

    return kern(x_t, table_t)


def kernel(x, table):
    x_t = x.T.astype(jnp.int32)
    table_t = table.T
    out_t = _embed_t(x_t, table_t)
    return out_t.transpose(2, 0, 1)
